# Initial kernel scaffold; baseline (speedup 1.0000x reference)
import jax
import jax.numpy as jnp
from jax.experimental import pallas as pl


def kernel(x, k):
    raise NotImplementedError("write your pallas kernel here")


# speedup vs baseline: 5.5715x; 5.5715x over previous
import functools

import jax
import jax.numpy as jnp
from jax import lax
from jax.experimental import pallas as pl
from jax.experimental.pallas import tpu as pltpu
from jax.experimental.pallas import tpu_sc as plsc

_L = 16


def _make_kwinners(B, C, HW, CH):
  assert C % CH == 0 and HW % _L == 0
  n_chunks = C // CH
  n_groups = HW // _L
  slab = C * HW
  chunk = CH * HW
  mesh = plsc.VectorSubcoreMesh(core_axis_name="c", subcore_axis_name="s")

  @functools.partial(
      pl.kernel,
      mesh=mesh,
      out_type=jax.ShapeDtypeStruct((B * C * HW,), jnp.float32),
      compiler_params=pltpu.CompilerParams(needs_layout_passes=False),
      scratch_types=[
          pltpu.VMEM((chunk,), jnp.float32),
          pltpu.VMEM((chunk,), jnp.float32),
          pltpu.VMEM((HW,), jnp.float32),
          pltpu.VMEM((HW,), jnp.int32),
      ],
  )
  def kw(x_hbm, o_hbm, ibuf, obuf, rmax, ridx):
    b = lax.axis_index("s") * 2 + lax.axis_index("c")
    base = b * slab
    neg_inf = jnp.full((_L,), -jnp.inf, jnp.float32)
    zero_i = jnp.zeros((_L,), jnp.int32)
    zero_f = jnp.zeros((_L,), jnp.float32)
    lane = lax.iota(jnp.int32, _L)

    def init_g(g, _):
      col = pl.ds(g * _L, _L)
      rmax[col] = neg_inf
      ridx[col] = zero_i
      return 0

    lax.fori_loop(0, n_groups, init_g, 0)

    def init_o(j, _):
      obuf[pl.ds(j * _L, _L)] = zero_f
      return 0

    lax.fori_loop(0, chunk // _L, init_o, 0)

    for ci in range(n_chunks):
      c0 = ci * CH
      pltpu.sync_copy(x_hbm.at[pl.ds(base + c0 * HW, chunk)], ibuf)

      def red_g(g, _):
        col = pl.ds(g * _L, _L)
        off = g * _L

        def red_r(r, carry):
          m, i = carry
          v = ibuf[pl.ds(r * HW + off, _L)]
          gt = v > m
          return jnp.where(gt, v, m), jnp.where(gt, c0 + r, i)

        m, i = lax.fori_loop(0, CH, red_r, (rmax[col], ridx[col]))
        rmax[col] = m
        ridx[col] = i
        return 0

      lax.fori_loop(0, n_groups, red_g, 0)

    for ci in range(n_chunks):
      c0 = ci * CH

      def scat_g(g, _):
        col = pl.ds(g * _L, _L)
        i = ridx[col]
        sel = (i >= c0) & (i < c0 + CH)
        flat = jnp.where(sel, (i - c0) * HW, 0) + (g * _L + lane)
        plsc.store_scatter(obuf, [flat], rmax[col], mask=sel)
        return 0

      lax.fori_loop(0, n_groups, scat_g, 0)
      pltpu.sync_copy(obuf, o_hbm.at[pl.ds(base + c0 * HW, chunk)])

      def zero_g(g, _):
        col = pl.ds(g * _L, _L)
        i = ridx[col]
        sel = (i >= c0) & (i < c0 + CH)
        flat = jnp.where(sel, (i - c0) * HW, 0) + (g * _L + lane)
        plsc.store_scatter(obuf, [flat], zero_f, mask=sel)
        return 0

      lax.fori_loop(0, n_groups, zero_g, 0)

  return kw


def kernel(x, k):
  B, C, H, W = x.shape
  xr = x.reshape(B * C * H * W)
  out = _make_kwinners(B, C, H * W, 48)(xr)
  return out.reshape(B, C, H, W)


# speedup vs baseline: 6.7751x; 1.2160x over previous
import functools

import jax
import jax.numpy as jnp
from jax import lax
from jax.experimental import pallas as pl
from jax.experimental.pallas import tpu as pltpu
from jax.experimental.pallas import tpu_sc as plsc

_L = 16


def _make_kwinners(B, C, HW, CH, CHO):
  assert C % CH == 0 and C % CHO == 0 and HW % _L == 0
  n_chunks = C // CH
  n_ochunks = C // CHO
  n_groups = HW // _L
  slab = C * HW
  chunk = CH * HW
  ochunk = CHO * HW
  mesh = plsc.VectorSubcoreMesh(core_axis_name="c", subcore_axis_name="s")

  @functools.partial(
      pl.kernel,
      mesh=mesh,
      out_type=jax.ShapeDtypeStruct((B * C * HW,), jnp.float32),
      compiler_params=pltpu.CompilerParams(needs_layout_passes=False),
      scratch_types=[
          pltpu.VMEM((chunk,), jnp.float32),
          pltpu.VMEM((chunk,), jnp.float32),
          pltpu.VMEM((ochunk,), jnp.float32),
          pltpu.VMEM((HW,), jnp.float32),
          pltpu.VMEM((HW,), jnp.int32),
          pltpu.SemaphoreType.DMA,
          pltpu.SemaphoreType.DMA,
      ],
  )
  def kw(x_hbm, o_hbm, ibuf_a, ibuf_b, obuf, rmax, ridx, sem_a, sem_b):
    b = lax.axis_index("s") * 2 + lax.axis_index("c")
    base = b * slab
    neg_inf = jnp.full((_L,), -jnp.inf, jnp.float32)
    zero_i = jnp.zeros((_L,), jnp.int32)
    zero_f = jnp.zeros((_L,), jnp.float32)
    lane = lax.iota(jnp.int32, _L)
    ibufs = (ibuf_a, ibuf_b)
    sems = (sem_a, sem_b)

    def init_g(g, _):
      col = pl.ds(g * _L, _L)
      rmax[col] = neg_inf
      ridx[col] = zero_i
      return 0

    lax.fori_loop(0, n_groups, init_g, 0)

    def init_o(j, _):
      obuf[pl.ds(j * _L, _L)] = zero_f
      return 0

    lax.fori_loop(0, ochunk // _L, init_o, 0)

    handles = [None, None]
    handles[0] = pltpu.async_copy(
        x_hbm.at[pl.ds(base, chunk)], ibuf_a, sem_a)
    for ci in range(n_chunks):
      cur = ci % 2
      nxt = 1 - cur
      if ci + 1 < n_chunks:
        handles[nxt] = pltpu.async_copy(
            x_hbm.at[pl.ds(base + (ci + 1) * chunk, chunk)],
            ibufs[nxt], sems[nxt])
      handles[cur].wait()
      buf = ibufs[cur]
      c0 = ci * CH

      def red_g(g, _):
        off = g * _L
        col = pl.ds(off, _L)
        m = rmax[col]
        i = ridx[col]
        for r in range(CH):
          v = buf[pl.ds(off + r * HW, _L)]
          gt = v > m
          m = jnp.where(gt, v, m)
          i = jnp.where(gt, c0 + r, i)
        rmax[col] = m
        ridx[col] = i
        return 0

      lax.fori_loop(0, n_groups, red_g, 0)

    for ci in range(n_ochunks):
      c0 = ci * CHO

      def scat_g(g, _):
        col = pl.ds(g * _L, _L)
        i = ridx[col]
        sel = (i >= c0) & (i < c0 + CHO)
        flat = jnp.where(sel, (i - c0) * HW, 0) + (g * _L + lane)
        plsc.store_scatter(obuf, [flat], rmax[col], mask=sel)
        return 0

      lax.fori_loop(0, n_groups, scat_g, 0)
      pltpu.sync_copy(obuf, o_hbm.at[pl.ds(base + c0 * HW, ochunk)])

      def zero_g(g, _):
        col = pl.ds(g * _L, _L)
        i = ridx[col]
        sel = (i >= c0) & (i < c0 + CHO)
        flat = jnp.where(sel, (i - c0) * HW, 0) + (g * _L + lane)
        plsc.store_scatter(obuf, [flat], zero_f, mask=sel)
        return 0

      lax.fori_loop(0, n_groups, zero_g, 0)

  return kw


def kernel(x, k):
  B, C, H, W = x.shape
  xr = x.reshape(B * C * H * W)
  out = _make_kwinners(B, C, H * W, 32, 32)(xr)
  return out.reshape(B, C, H, W)
